# baseline (speedup 1.0000x reference)
import jax
import jax.numpy as jnp
from jax.experimental import pallas as pl
from jax.experimental.pallas import tpu as pltpu

_N, _E, _T, _D = 19, 342, 1025, 512
_NEG = -1e30


def _leaky(z):
    return jnp.where(z > 0, z, 0.2 * z)


def _edge_stage(xl, xr, srcf, dstf, dmask, a_row, b_row):
    xls = jax.lax.dot_general(srcf, xl, (((0,), (0,)), ((), ())),
                              preferred_element_type=jnp.float32)
    xrd = jax.lax.dot_general(dstf, xr, (((0,), (0,)), ((), ())),
                              preferred_element_type=jnp.float32)
    he = _leaky(xls + xrd)
    e_row = jax.lax.dot_general(a_row, he, (((1,), (1,)), ((), ())),
                                preferred_element_type=jnp.float32)
    eb = jnp.broadcast_to(e_row, (_N, _E))
    m = jnp.max(jnp.where(dmask, eb, _NEG), axis=1, keepdims=True)
    mdst = jnp.max(jnp.where(dmask, jnp.broadcast_to(m, (_N, _E)), _NEG),
                   axis=0, keepdims=True)
    ex = jnp.exp(jnp.minimum(e_row - mdst, 0.0))
    s = jnp.sum(jnp.where(dmask, jnp.broadcast_to(ex, (_N, _E)), 0.0),
                axis=1, keepdims=True)
    sdst = jnp.sum(jnp.where(dmask, jnp.broadcast_to(s, (_N, _E)), 0.0),
                   axis=0, keepdims=True)
    alpha = ex / (sdst + 1e-16)
    aw = jnp.where(dmask, jnp.broadcast_to(alpha, (_N, _E)), 0.0)
    adj = jax.lax.dot_general(aw, srcf, (((1,), (1,)), ((), ())),
                              preferred_element_type=jnp.float32)
    h = jnp.dot(adj, xl, preferred_element_type=jnp.float32) + b_row
    return h, alpha


def _fused(ei_ref, x_ref, a1_ref, b1_ref, a2_ref, b2_ref,
           mmse_ref, wm_ref, bm_ref,
           wl1_hbm, wr1_hbm, wl2_hbm, wr2_hbm,
           dec_ref, al_ref,
           w1_buf, w2_buf, w3_buf, w4_buf, s1, s2, s3, s4):
    c1 = pltpu.make_async_copy(wl1_hbm, w1_buf, s1)
    c2 = pltpu.make_async_copy(wr1_hbm, w2_buf, s2)
    c3 = pltpu.make_async_copy(wl2_hbm, w3_buf, s3)
    c4 = pltpu.make_async_copy(wr2_hbm, w4_buf, s4)
    c1.start()
    c2.start()
    c3.start()
    c4.start()
    src_row = ei_ref[0:1, :]
    dst_row = ei_ref[1:2, :]
    ion = jax.lax.broadcasted_iota(jnp.int32, (_N, _E), 0)
    srcf = (ion == src_row).astype(jnp.float32)
    dmask = ion == dst_row
    dstf = dmask.astype(jnp.float32)
    x = x_ref[:]
    c1.wait()
    xl1 = jnp.dot(x, w1_buf[:], preferred_element_type=jnp.float32)
    c2.wait()
    xr1 = jnp.dot(x, w2_buf[:], preferred_element_type=jnp.float32)
    h1, alpha1 = _edge_stage(xl1, xr1, srcf, dstf, dmask,
                             a1_ref[:].reshape(1, _D), b1_ref[:].reshape(1, _D))
    c3.wait()
    xl2 = jnp.dot(h1, w3_buf[:], preferred_element_type=jnp.float32)
    c4.wait()
    xr2 = jnp.dot(h1, w4_buf[:], preferred_element_type=jnp.float32)
    h2, _ = _edge_stage(xl2, xr2, srcf, dstf, dmask,
                        a2_ref[:].reshape(1, _D), b2_ref[:].reshape(1, _D))
    gf = h2 + (mmse_ref[0] * wm_ref[:] + bm_ref[:].reshape(1, _D))
    dec = jax.lax.dot_general(gf, gf, (((1,), (1,)), ((), ())),
                              preferred_element_type=jnp.float32)
    dec_ref[:] = jax.nn.sigmoid(dec)
    al_ref[:] = alpha1


def kernel(x, edge_index, mmse, Wl1, Wr1, a1, b1, Wl2, Wr2, a2, b2, Wm, bm,
           W11, b11, W12, b12, W21, b21, W22, b22, Wp, bp):
    vmem = pl.BlockSpec(memory_space=pltpu.VMEM)
    hbm = pl.BlockSpec(memory_space=pl.ANY)
    dec, al = pl.pallas_call(
        _fused,
        in_specs=[vmem] * 9 + [hbm] * 4,
        out_shape=[jax.ShapeDtypeStruct((_N, _N), jnp.float32),
                   jax.ShapeDtypeStruct((1, _E), jnp.float32)],
        scratch_shapes=[
            pltpu.VMEM((_T, _D), jnp.float32),
            pltpu.VMEM((_T, _D), jnp.float32),
            pltpu.VMEM((_D, _D), jnp.float32),
            pltpu.VMEM((_D, _D), jnp.float32),
            pltpu.SemaphoreType.DMA,
            pltpu.SemaphoreType.DMA,
            pltpu.SemaphoreType.DMA,
            pltpu.SemaphoreType.DMA,
        ],
    )(edge_index, x, a1, b1, a2, b2, mmse, Wm, bm, Wl1, Wr1, Wl2, Wr2)
    return dec, al[0]


# speedup vs baseline: 1.1041x; 1.1041x over previous
import jax
import jax.numpy as jnp
from jax.experimental import pallas as pl
from jax.experimental.pallas import tpu as pltpu

_N, _E, _T, _D = 19, 342, 1025, 512
_NEG = -1e30


def _leaky(z):
    return jnp.where(z > 0, z, 0.2 * z)


def _edge_stage(xl, xr, srcf, dstf, dmask, a_row, b_row):
    xls = jax.lax.dot_general(srcf, xl, (((0,), (0,)), ((), ())),
                              preferred_element_type=jnp.float32)
    xrd = jax.lax.dot_general(dstf, xr, (((0,), (0,)), ((), ())),
                              preferred_element_type=jnp.float32)
    he = _leaky(xls + xrd)
    e_row = jax.lax.dot_general(a_row, he, (((1,), (1,)), ((), ())),
                                preferred_element_type=jnp.float32)
    eb = jnp.broadcast_to(e_row, (_N, _E))
    m = jnp.max(jnp.where(dmask, eb, _NEG), axis=1, keepdims=True)
    mdst = jnp.max(jnp.where(dmask, jnp.broadcast_to(m, (_N, _E)), _NEG),
                   axis=0, keepdims=True)
    ex = jnp.exp(jnp.minimum(e_row - mdst, 0.0))
    s = jnp.sum(jnp.where(dmask, jnp.broadcast_to(ex, (_N, _E)), 0.0),
                axis=1, keepdims=True)
    sdst = jnp.sum(jnp.where(dmask, jnp.broadcast_to(s, (_N, _E)), 0.0),
                   axis=0, keepdims=True)
    alpha = ex / (sdst + 1e-16)
    aw = jnp.where(dmask, jnp.broadcast_to(alpha, (_N, _E)), 0.0)
    adj = jax.lax.dot_general(aw, srcf, (((1,), (1,)), ((), ())),
                              preferred_element_type=jnp.float32)
    h = jnp.dot(adj, xl, preferred_element_type=jnp.float32) + b_row
    return h, alpha


def _fused(ei_ref, x_ref, a1_ref, b1_ref, a2_ref, b2_ref,
           mmse_ref, wm_ref, bm_ref, wl1_ref, wr1_ref,
           wl2_hbm, wr2_hbm,
           dec_ref, al_ref,
           w3_buf, w4_buf, s3, s4):
    c3 = pltpu.make_async_copy(wl2_hbm, w3_buf, s3)
    c4 = pltpu.make_async_copy(wr2_hbm, w4_buf, s4)
    c3.start()
    c4.start()
    src_row = ei_ref[0:1, :]
    dst_row = ei_ref[1:2, :]
    ion = jax.lax.broadcasted_iota(jnp.int32, (_N, _E), 0)
    srcf = (ion == src_row).astype(jnp.float32)
    dmask = ion == dst_row
    dstf = dmask.astype(jnp.float32)
    x = x_ref[:]
    xl1 = jnp.dot(x, wl1_ref[:], preferred_element_type=jnp.float32)
    xr1 = jnp.dot(x, wr1_ref[:], preferred_element_type=jnp.float32)
    h1, alpha1 = _edge_stage(xl1, xr1, srcf, dstf, dmask,
                             a1_ref[:].reshape(1, _D), b1_ref[:].reshape(1, _D))
    c3.wait()
    xl2 = jnp.dot(h1, w3_buf[:], preferred_element_type=jnp.float32)
    c4.wait()
    xr2 = jnp.dot(h1, w4_buf[:], preferred_element_type=jnp.float32)
    h2, _ = _edge_stage(xl2, xr2, srcf, dstf, dmask,
                        a2_ref[:].reshape(1, _D), b2_ref[:].reshape(1, _D))
    gf = h2 + (mmse_ref[0] * wm_ref[:] + bm_ref[:].reshape(1, _D))
    dec = jax.lax.dot_general(gf, gf, (((1,), (1,)), ((), ())),
                              preferred_element_type=jnp.float32)
    dec_ref[:] = jax.nn.sigmoid(dec)
    al_ref[:] = alpha1


def kernel(x, edge_index, mmse, Wl1, Wr1, a1, b1, Wl2, Wr2, a2, b2, Wm, bm,
           W11, b11, W12, b12, W21, b21, W22, b22, Wp, bp):
    vmem = pl.BlockSpec(memory_space=pltpu.VMEM)
    hbm = pl.BlockSpec(memory_space=pl.ANY)
    dec, al = pl.pallas_call(
        _fused,
        in_specs=[vmem] * 11 + [hbm] * 2,
        out_shape=[jax.ShapeDtypeStruct((_N, _N), jnp.float32),
                   jax.ShapeDtypeStruct((1, _E), jnp.float32)],
        scratch_shapes=[
            pltpu.VMEM((_D, _D), jnp.float32),
            pltpu.VMEM((_D, _D), jnp.float32),
            pltpu.SemaphoreType.DMA,
            pltpu.SemaphoreType.DMA,
        ],
    )(edge_index, x, a1, b1, a2, b2, mmse, Wm, bm, Wl1, Wr1, Wl2, Wr2)
    return dec, al[0]
